# initial kernel scaffold (unmeasured)
import jax
import jax.numpy as jnp
from jax import lax
from jax.experimental import pallas as pl
from jax.experimental.pallas import tpu as pltpu

N_DEV = 8
SQ = 2048
DMODEL = 1024
HQ = 8
DH = 128
SKV_LOC = 2048
BLK = 64
NRES = 4
NJ = SQ // (BLK * NRES)
SCALE = 0.08838834764831843


def _attn_body(x_ref, wq_ref, k_ref, v_ref, o_ref, m_ref, l_ref):
    q = jnp.dot(x_ref[...], wq_ref[...], preferred_element_type=jnp.float32)
    for r in range(NRES):
        qr = jnp.concatenate(
            [q[BLK * (r + NRES * j):BLK * (r + NRES * j) + BLK, :] for j in range(NJ)],
            axis=0,
        )
        kr = jnp.concatenate(
            [k_ref[pl.ds(BLK * (r + NRES * j), BLK), 0, :] for j in range(NJ)], axis=0
        )
        vr = jnp.concatenate(
            [v_ref[pl.ds(BLK * (r + NRES * j), BLK), 0, :] for j in range(NJ)], axis=0
        )
        s = lax.dot_general(
            qr, kr, (((1,), (1,)), ((), ())), preferred_element_type=jnp.float32
        ) * SCALE
        mr = jnp.max(s, axis=1)
        p = jnp.exp(s - mr[:, None])
        lr = jnp.sum(p, axis=1)
        orr = jnp.dot(p, vr, preferred_element_type=jnp.float32)
        for j in range(NJ):
            sl = pl.ds(BLK * (r + NRES * j), BLK)
            o_ref[sl, :] = orr[BLK * j:BLK * (j + 1), :]
            m_ref[sl, :] = mr[BLK * j:BLK * (j + 1)][:, None]
            l_ref[sl, :] = lr[BLK * j:BLK * (j + 1)][:, None]


def _combine_body(
    o_ref, m_ref, l_ref, wo_ref, out_ref,
    o_comm, ml_comm, acc_o, acc_m, acc_l,
    o_ssem, o_rsem, ml_ssem, ml_rsem,
):
    my = lax.axis_index("i")
    left = lax.rem(my - 1 + N_DEV, N_DEV)
    right = lax.rem(my + 1, N_DEV)

    barrier = pltpu.get_barrier_semaphore()
    for nbr in (left, right):
        pl.semaphore_signal(
            barrier, inc=1, device_id=(nbr,), device_id_type=pl.DeviceIdType.MESH
        )
    pl.semaphore_wait(barrier, 2)

    acc_o[...] = o_ref[...]
    acc_m[...] = m_ref[...]
    acc_l[...] = l_ref[...]
    o_comm[0, :, :] = o_ref[...]
    ml_comm[0, :, :] = jnp.concatenate([m_ref[...], l_ref[...]], axis=1)

    for h in range(N_DEV - 1):
        snd = h % 2
        rcv = (h + 1) % 2
        o_rdma = pltpu.make_async_remote_copy(
            src_ref=o_comm.at[snd],
            dst_ref=o_comm.at[rcv],
            send_sem=o_ssem.at[snd],
            recv_sem=o_rsem.at[rcv],
            device_id=(right,),
            device_id_type=pl.DeviceIdType.MESH,
        )
        ml_rdma = pltpu.make_async_remote_copy(
            src_ref=ml_comm.at[snd],
            dst_ref=ml_comm.at[rcv],
            send_sem=ml_ssem.at[snd],
            recv_sem=ml_rsem.at[rcv],
            device_id=(right,),
            device_id_type=pl.DeviceIdType.MESH,
        )
        o_rdma.start()
        ml_rdma.start()
        o_rdma.wait()
        ml_rdma.wait()

        m_in = ml_comm[rcv, :, 0:HQ]
        l_in = ml_comm[rcv, :, HQ:2 * HQ]
        m_new = jnp.maximum(acc_m[...], m_in)
        a = jnp.exp(acc_m[...] - m_new)
        b = jnp.exp(m_in - m_new)
        acc_l[...] = acc_l[...] * a + l_in * b
        for hh in range(HQ):
            cs = pl.ds(hh * DH, DH)
            acc_o[:, cs] = (
                acc_o[:, cs] * a[:, hh][:, None]
                + o_comm[rcv, :, cs] * b[:, hh][:, None]
            )
        acc_m[...] = m_new

    for hh in range(HQ):
        cs = pl.ds(hh * DH, DH)
        acc_o[:, cs] = acc_o[:, cs] / acc_l[:, hh][:, None]

    out_ref[...] = jnp.dot(
        acc_o[...], wo_ref[...], preferred_element_type=jnp.float32
    )


def kernel(x, Wq, K_ext, V_ext, Wo):
    x2 = x.reshape(SQ, DMODEL)
    k = K_ext.reshape(SKV_LOC, HQ, DH)
    v = V_ext.reshape(SKV_LOC, HQ, DH)

    o, m, l = pl.pallas_call(
        _attn_body,
        grid=(HQ,),
        in_specs=[
            pl.BlockSpec((SQ, DMODEL), lambda h: (0, 0)),
            pl.BlockSpec((DMODEL, DH), lambda h: (0, h)),
            pl.BlockSpec((SKV_LOC, 1, DH), lambda h: (0, h, 0)),
            pl.BlockSpec((SKV_LOC, 1, DH), lambda h: (0, h, 0)),
        ],
        out_shape=[
            jax.ShapeDtypeStruct((SQ, DMODEL), jnp.float32),
            jax.ShapeDtypeStruct((SQ, HQ), jnp.float32),
            jax.ShapeDtypeStruct((SQ, HQ), jnp.float32),
        ],
        out_specs=[
            pl.BlockSpec((SQ, DH), lambda h: (0, h)),
            pl.BlockSpec((SQ, 1), lambda h: (0, h)),
            pl.BlockSpec((SQ, 1), lambda h: (0, h)),
        ],
    )(x2, Wq, k, v)

    out = pl.pallas_call(
        _combine_body,
        in_specs=[
            pl.BlockSpec(memory_space=pltpu.VMEM),
            pl.BlockSpec(memory_space=pltpu.VMEM),
            pl.BlockSpec(memory_space=pltpu.VMEM),
            pl.BlockSpec(memory_space=pltpu.VMEM),
        ],
        out_shape=jax.ShapeDtypeStruct((SQ, DMODEL), jnp.float32),
        out_specs=pl.BlockSpec(memory_space=pltpu.VMEM),
        scratch_shapes=[
            pltpu.VMEM((2, SQ, DMODEL), jnp.float32),
            pltpu.VMEM((2, SQ, 2 * HQ), jnp.float32),
            pltpu.VMEM((SQ, DMODEL), jnp.float32),
            pltpu.VMEM((SQ, HQ), jnp.float32),
            pltpu.VMEM((SQ, HQ), jnp.float32),
            pltpu.SemaphoreType.DMA((2,)),
            pltpu.SemaphoreType.DMA((2,)),
            pltpu.SemaphoreType.DMA((2,)),
            pltpu.SemaphoreType.DMA((2,)),
        ],
        compiler_params=pltpu.CompilerParams(collective_id=0),
    )(o, m, l, Wo)

    return out.reshape(1, SQ, DMODEL)


# baseline (device time: 840645 ns/iter reference)
import jax
import jax.numpy as jnp
from jax import lax
from jax.experimental import pallas as pl
from jax.experimental.pallas import tpu as pltpu

N_DEV = 8
SQ = 2048
DMODEL = 1024
HQ = 8
DH = 128
SKV_LOC = 2048
BLK = 64
NRES = 4
NJ = SQ // (BLK * NRES)
RROWS = SQ // NRES
SCALE = 0.08838834764831843

PERM = [r + NRES * j for r in range(NRES) for j in range(NJ)]


def _attn_body(x_ref, wq_ref, k_ref, v_ref, o_ref, ml_ref):
    q_all = jnp.dot(x_ref[...], wq_ref[...], preferred_element_type=jnp.float32)
    for hh in range(HQ):
        hs = pl.ds(hh * DH, DH)
        for r in range(NRES):
            rs = pl.ds(r * RROWS, RROWS)
            qr = q_all[r * RROWS:(r + 1) * RROWS, hh * DH:(hh + 1) * DH]
            kr = k_ref[rs, hs]
            vr = v_ref[rs, hs]
            s = lax.dot_general(
                qr, kr, (((1,), (1,)), ((), ())), preferred_element_type=jnp.float32
            ) * SCALE
            mr = jnp.max(s, axis=1)
            p = jnp.exp(s - mr[:, None])
            lr = jnp.sum(p, axis=1)
            orr = jnp.dot(p, vr, preferred_element_type=jnp.float32)
            o_ref[rs, hs] = orr
            ml_ref[rs, hh:hh + 1] = mr[:, None]
            ml_ref[rs, HQ + hh:HQ + hh + 1] = lr[:, None]


def _combine_body(
    o_ref, ml_ref, out_o, out_ml,
    o_comm, ml_comm,
    o_ssem, o_rsem, ml_ssem, ml_rsem,
):
    my = lax.axis_index("i")
    left = lax.rem(my - 1 + N_DEV, N_DEV)
    right = lax.rem(my + 1, N_DEV)

    barrier = pltpu.get_barrier_semaphore()
    for nbr in (left, right):
        pl.semaphore_signal(
            barrier, inc=1, device_id=(nbr,), device_id_type=pl.DeviceIdType.MESH
        )
    pl.semaphore_wait(barrier, 2)

    out_o[...] = o_ref[...]
    out_ml[...] = ml_ref[...]
    o_comm[0, :, :] = o_ref[...]
    ml_comm[0, :, :] = ml_ref[...]

    for h in range(N_DEV - 1):
        snd = h % 2
        rcv = (h + 1) % 2
        o_rdma = pltpu.make_async_remote_copy(
            src_ref=o_comm.at[snd],
            dst_ref=o_comm.at[rcv],
            send_sem=o_ssem.at[snd],
            recv_sem=o_rsem.at[rcv],
            device_id=(right,),
            device_id_type=pl.DeviceIdType.MESH,
        )
        ml_rdma = pltpu.make_async_remote_copy(
            src_ref=ml_comm.at[snd],
            dst_ref=ml_comm.at[rcv],
            send_sem=ml_ssem.at[snd],
            recv_sem=ml_rsem.at[rcv],
            device_id=(right,),
            device_id_type=pl.DeviceIdType.MESH,
        )
        o_rdma.start()
        ml_rdma.start()
        o_rdma.wait()
        ml_rdma.wait()

        m_acc = out_ml[:, 0:HQ]
        l_acc = out_ml[:, HQ:2 * HQ]
        m_in = ml_comm[rcv, :, 0:HQ]
        l_in = ml_comm[rcv, :, HQ:2 * HQ]
        m_new = jnp.maximum(m_acc, m_in)
        a = jnp.exp(m_acc - m_new)
        b = jnp.exp(m_in - m_new)
        out_ml[:, HQ:2 * HQ] = l_acc * a + l_in * b
        out_ml[:, 0:HQ] = m_new
        for hh in range(HQ):
            hs = pl.ds(hh * DH, DH)
            out_o[:, hs] = (
                out_o[:, hs] * a[:, hh:hh + 1]
                + o_comm[rcv, :, hs] * b[:, hh:hh + 1]
            )


def _project_body(o_ref, ml_ref, wo_ref, out_ref):
    for r in range(NRES):
        rs = pl.ds(r * RROWS, RROWS)
        ctx = jnp.concatenate(
            [
                o_ref[rs, pl.ds(hh * DH, DH)]
                / ml_ref[rs, HQ + hh:HQ + hh + 1]
                for hh in range(HQ)
            ],
            axis=1,
        )
        outp = jnp.dot(ctx, wo_ref[...], preferred_element_type=jnp.float32)
        for j in range(NJ):
            blk = r + NRES * j
            out_ref[pl.ds(blk * BLK, BLK), :] = outp[j * BLK:(j + 1) * BLK, :]


def kernel(x, Wq, K_ext, V_ext, Wo):
    perm = jnp.asarray(PERM, dtype=jnp.int32)
    x_p = x.reshape(SQ // BLK, BLK, DMODEL)[perm].reshape(SQ, DMODEL)
    k_p = K_ext.reshape(SKV_LOC // BLK, BLK, HQ * DH)[perm].reshape(SKV_LOC, HQ * DH)
    v_p = V_ext.reshape(SKV_LOC // BLK, BLK, HQ * DH)[perm].reshape(SKV_LOC, HQ * DH)

    o, ml = pl.pallas_call(
        _attn_body,
        in_specs=[pl.BlockSpec(memory_space=pltpu.VMEM)] * 4,
        out_shape=[
            jax.ShapeDtypeStruct((SQ, DMODEL), jnp.float32),
            jax.ShapeDtypeStruct((SQ, 2 * HQ), jnp.float32),
        ],
        out_specs=[pl.BlockSpec(memory_space=pltpu.VMEM)] * 2,
        compiler_params=pltpu.CompilerParams(vmem_limit_bytes=100 * 1024 * 1024),
    )(x_p, Wq, k_p, v_p)

    o_c, ml_c = pl.pallas_call(
        _combine_body,
        in_specs=[pl.BlockSpec(memory_space=pltpu.VMEM)] * 2,
        out_shape=[
            jax.ShapeDtypeStruct((SQ, DMODEL), jnp.float32),
            jax.ShapeDtypeStruct((SQ, 2 * HQ), jnp.float32),
        ],
        out_specs=[pl.BlockSpec(memory_space=pltpu.VMEM)] * 2,
        scratch_shapes=[
            pltpu.VMEM((2, SQ, DMODEL), jnp.float32),
            pltpu.VMEM((2, SQ, 2 * HQ), jnp.float32),
            pltpu.SemaphoreType.DMA((2,)),
            pltpu.SemaphoreType.DMA((2,)),
            pltpu.SemaphoreType.DMA((2,)),
            pltpu.SemaphoreType.DMA((2,)),
        ],
        compiler_params=pltpu.CompilerParams(
            collective_id=0, vmem_limit_bytes=100 * 1024 * 1024
        ),
    )(o, ml)

    out = pl.pallas_call(
        _project_body,
        in_specs=[pl.BlockSpec(memory_space=pltpu.VMEM)] * 3,
        out_shape=jax.ShapeDtypeStruct((SQ, DMODEL), jnp.float32),
        out_specs=pl.BlockSpec(memory_space=pltpu.VMEM),
        compiler_params=pltpu.CompilerParams(vmem_limit_bytes=100 * 1024 * 1024),
    )(o_c, ml_c, Wo)

    return out.reshape(1, SQ, DMODEL)


# device time: 267480 ns/iter; 3.1428x vs baseline; 3.1428x over previous
import jax
import jax.numpy as jnp
from jax import lax
from jax.experimental import pallas as pl
from jax.experimental.pallas import tpu as pltpu

N_DEV = 8
SQ = 2048
DMODEL = 1024
HQ = 8
DH = 128
SKV_LOC = 2048
BLK = 64
NRES = 4
NJ = SQ // (BLK * NRES)
RROWS = SQ // NRES
CH = SQ // N_DEV
SCALE = 0.08838834764831843

PERM = [r + NRES * j for r in range(NRES) for j in range(NJ)]


def _attn_body(x_ref, wq_ref, k_ref, v_ref, o_ref, ml_ref):
    q_all = jnp.dot(x_ref[...], wq_ref[...], preferred_element_type=jnp.float32)
    for hh in range(HQ):
        hs = pl.ds(hh * DH, DH)
        for r in range(NRES):
            rs = pl.ds(r * RROWS, RROWS)
            qr = q_all[r * RROWS:(r + 1) * RROWS, hh * DH:(hh + 1) * DH]
            kr = k_ref[rs, hs]
            vr = v_ref[rs, hs]
            s = lax.dot_general(
                qr, kr, (((1,), (1,)), ((), ())), preferred_element_type=jnp.float32
            ) * SCALE
            mr = jnp.max(s, axis=1)
            p = jnp.exp(s - mr[:, None])
            lr = jnp.sum(p, axis=1)
            orr = jnp.dot(p, vr, preferred_element_type=jnp.float32)
            o_ref[rs, hs] = orr
            ml_ref[rs, hh:hh + 1] = mr[:, None]
            ml_ref[rs, HQ + hh:HQ + hh + 1] = lr[:, None]


def _store_chunk_unpermuted(out_ref, chunk, c):
    for i in range(CH // BLK):
        p = c * (CH // BLK) + i
        blk = lax.rem(p, NJ) * NRES + lax.div(p, NJ)
        out_ref[pl.ds(blk * BLK, BLK), :] = chunk[i * BLK:(i + 1) * BLK, :]


def _combine_body(
    o_ref, ml_ref, wo_ref, out_ref,
    acc_o, acc_ml, o_comm, ml_comm, ag_comm,
    o_ssem, o_rsem, ml_ssem, ml_rsem, ag_ssem, ag_rsem,
):
    my = lax.axis_index("i")
    left = lax.rem(my - 1 + N_DEV, N_DEV)
    right = lax.rem(my + 1, N_DEV)

    barrier = pltpu.get_barrier_semaphore()
    for nbr in (left, right):
        pl.semaphore_signal(
            barrier, inc=1, device_id=(nbr,), device_id_type=pl.DeviceIdType.MESH
        )
    pl.semaphore_wait(barrier, 2)

    acc_o[...] = o_ref[...]
    acc_ml[...] = ml_ref[...]

    for t in range(N_DEV - 1):
        snd = t % 2
        rcv = (t + 1) % 2
        c_send = lax.rem(my - t + N_DEV, N_DEV)
        c_recv = lax.rem(my - t - 1 + 2 * N_DEV, N_DEV)
        o_comm[snd] = acc_o[pl.ds(c_send * CH, CH), :]
        ml_comm[snd] = acc_ml[pl.ds(c_send * CH, CH), :]
        o_rdma = pltpu.make_async_remote_copy(
            src_ref=o_comm.at[snd], dst_ref=o_comm.at[rcv],
            send_sem=o_ssem.at[snd], recv_sem=o_rsem.at[rcv],
            device_id=(right,), device_id_type=pl.DeviceIdType.MESH,
        )
        ml_rdma = pltpu.make_async_remote_copy(
            src_ref=ml_comm.at[snd], dst_ref=ml_comm.at[rcv],
            send_sem=ml_ssem.at[snd], recv_sem=ml_rsem.at[rcv],
            device_id=(right,), device_id_type=pl.DeviceIdType.MESH,
        )
        o_rdma.start()
        ml_rdma.start()
        o_rdma.wait()
        ml_rdma.wait()

        rs = pl.ds(c_recv * CH, CH)
        m_acc = acc_ml[rs, 0:HQ]
        l_acc = acc_ml[rs, HQ:2 * HQ]
        m_in = ml_comm[rcv, :, 0:HQ]
        l_in = ml_comm[rcv, :, HQ:2 * HQ]
        m_new = jnp.maximum(m_acc, m_in)
        a = jnp.exp(m_acc - m_new)
        b = jnp.exp(m_in - m_new)
        acc_ml[rs, HQ:2 * HQ] = l_acc * a + l_in * b
        acc_ml[rs, 0:HQ] = m_new
        for hh in range(HQ):
            hs = pl.ds(hh * DH, DH)
            acc_o[rs, hs] = (
                acc_o[rs, hs] * a[:, hh:hh + 1]
                + o_comm[rcv, :, hs] * b[:, hh:hh + 1]
            )

    q = lax.rem(my + 1, N_DEV)
    qs = pl.ds(q * CH, CH)
    ctx = jnp.concatenate(
        [
            acc_o[qs, pl.ds(hh * DH, DH)] / acc_ml[qs, HQ + hh:HQ + hh + 1]
            for hh in range(HQ)
        ],
        axis=1,
    )
    mine = jnp.dot(ctx, wo_ref[...], preferred_element_type=jnp.float32)
    _store_chunk_unpermuted(out_ref, mine, q)
    ag_comm[0] = mine

    for h in range(N_DEV - 1):
        snd = h % 2
        rcv = (h + 1) % 2
        ag_rdma = pltpu.make_async_remote_copy(
            src_ref=ag_comm.at[snd], dst_ref=ag_comm.at[rcv],
            send_sem=ag_ssem.at[snd], recv_sem=ag_rsem.at[rcv],
            device_id=(right,), device_id_type=pl.DeviceIdType.MESH,
        )
        ag_rdma.start()
        ag_rdma.wait()
        c_recv = lax.rem(my - h + N_DEV, N_DEV)
        _store_chunk_unpermuted(out_ref, ag_comm[rcv], c_recv)


def kernel(x, Wq, K_ext, V_ext, Wo):
    perm = jnp.asarray(PERM, dtype=jnp.int32)
    x_p = x.reshape(SQ // BLK, BLK, DMODEL)[perm].reshape(SQ, DMODEL)
    k_p = K_ext.reshape(SKV_LOC // BLK, BLK, HQ * DH)[perm].reshape(SKV_LOC, HQ * DH)
    v_p = V_ext.reshape(SKV_LOC // BLK, BLK, HQ * DH)[perm].reshape(SKV_LOC, HQ * DH)

    o, ml = pl.pallas_call(
        _attn_body,
        in_specs=[pl.BlockSpec(memory_space=pltpu.VMEM)] * 4,
        out_shape=[
            jax.ShapeDtypeStruct((SQ, DMODEL), jnp.float32),
            jax.ShapeDtypeStruct((SQ, 2 * HQ), jnp.float32),
        ],
        out_specs=[pl.BlockSpec(memory_space=pltpu.VMEM)] * 2,
        compiler_params=pltpu.CompilerParams(vmem_limit_bytes=100 * 1024 * 1024),
    )(x_p, Wq, k_p, v_p)

    out = pl.pallas_call(
        _combine_body,
        in_specs=[pl.BlockSpec(memory_space=pltpu.VMEM)] * 3,
        out_shape=jax.ShapeDtypeStruct((SQ, DMODEL), jnp.float32),
        out_specs=pl.BlockSpec(memory_space=pltpu.VMEM),
        scratch_shapes=[
            pltpu.VMEM((SQ, DMODEL), jnp.float32),
            pltpu.VMEM((SQ, 2 * HQ), jnp.float32),
            pltpu.VMEM((2, CH, DMODEL), jnp.float32),
            pltpu.VMEM((2, CH, 2 * HQ), jnp.float32),
            pltpu.VMEM((2, CH, DMODEL), jnp.float32),
            pltpu.SemaphoreType.DMA((2,)),
            pltpu.SemaphoreType.DMA((2,)),
            pltpu.SemaphoreType.DMA((2,)),
            pltpu.SemaphoreType.DMA((2,)),
            pltpu.SemaphoreType.DMA((2,)),
            pltpu.SemaphoreType.DMA((2,)),
        ],
        compiler_params=pltpu.CompilerParams(
            collective_id=0, vmem_limit_bytes=100 * 1024 * 1024
        ),
    )(o, ml, Wo)

    return out.reshape(1, SQ, DMODEL)


# device time: 194707 ns/iter; 4.3175x vs baseline; 1.3738x over previous
import jax
import jax.numpy as jnp
from jax import lax
from jax.experimental import pallas as pl
from jax.experimental.pallas import tpu as pltpu

N_DEV = 8
SQ = 2048
DMODEL = 1024
HQ = 8
DH = 128
SKV_LOC = 2048
BLK = 64
NRES = 4
NJ = SQ // (BLK * NRES)
RROWS = SQ // NRES
CH = SQ // N_DEV
SCALE = 0.08838834764831843

PERM = [r + NRES * j for r in range(NRES) for j in range(NJ)]


def _attn_body(x_ref, wq_ref, k_ref, v_ref, o_ref, ml_ref):
    q_all = jnp.dot(x_ref[...], wq_ref[...], preferred_element_type=jnp.float32)
    for hh in range(HQ):
        hs = pl.ds(hh * DH, DH)
        for r in range(NRES):
            rs = pl.ds(r * RROWS, RROWS)
            qr = q_all[r * RROWS:(r + 1) * RROWS, hh * DH:(hh + 1) * DH]
            kr = k_ref[rs, hs]
            vr = v_ref[rs, hs]
            s = lax.dot_general(
                qr, kr, (((1,), (1,)), ((), ())), preferred_element_type=jnp.float32
            ) * SCALE
            mr = jnp.max(s, axis=1)
            p = jnp.exp(s - mr[:, None])
            lr = jnp.sum(p, axis=1)
            orr = jnp.dot(p, vr, preferred_element_type=jnp.float32)
            o_ref[rs, hs] = orr
            ml_ref[rs, hh:hh + 1] = mr[:, None]
            ml_ref[rs, HQ + hh:HQ + hh + 1] = lr[:, None]


def _store_chunk_unpermuted(out_ref, chunk, c):
    for i in range(CH // BLK):
        p = c * (CH // BLK) + i
        blk = lax.rem(p, NJ) * NRES + lax.div(p, NJ)
        out_ref[pl.ds(blk * BLK, BLK), :] = chunk[i * BLK:(i + 1) * BLK, :]


def _store_half_chunk(out_ref, chunk, c, co):
    for i in range(CH // BLK):
        p = c * (CH // BLK) + i
        blk = lax.rem(p, NJ) * NRES + lax.div(p, NJ)
        out_ref[pl.ds(blk * BLK, BLK), pl.ds(co, HCOL)] = (
            chunk[i * BLK:(i + 1) * BLK, :]
        )


HCOL = DMODEL // 2
HH = HQ // 2


def _combine_body(
    o_ref, ml_ref, wo_ref, out_ref,
    acc_o, acc_ml,
    o_cw, o_ccw, ml_cw, ml_ccw, ag_cw, ag_ccw,
    o_cw_s, o_cw_r, o_ccw_s, o_ccw_r,
    ml_cw_s, ml_cw_r, ml_ccw_s, ml_ccw_r,
    ag_cw_s, ag_cw_r, ag_ccw_s, ag_ccw_r,
):
    my = lax.axis_index("i")
    left = lax.rem(my - 1 + N_DEV, N_DEV)
    right = lax.rem(my + 1, N_DEV)

    barrier = pltpu.get_barrier_semaphore()
    for nbr in (left, right):
        pl.semaphore_signal(
            barrier, inc=1, device_id=(nbr,), device_id_type=pl.DeviceIdType.MESH
        )
    pl.semaphore_wait(barrier, 2)

    acc_o[...] = o_ref[...]
    acc_ml[...] = ml_ref[...]

    dirs = [
        (right, 0, 0, o_cw, ml_cw, o_cw_s, o_cw_r, ml_cw_s, ml_cw_r),
        (left, HCOL, HH, o_ccw, ml_ccw, o_ccw_s, o_ccw_r, ml_ccw_s, ml_ccw_r),
    ]
    for t in range(N_DEV - 1):
        snd = t % 2
        rcv = (t + 1) % 2
        c_send = [
            lax.rem(my - t + N_DEV, N_DEV),
            lax.rem(my + t + 2, N_DEV),
        ]
        c_recv = [
            lax.rem(my - t - 1 + 2 * N_DEV, N_DEV),
            lax.rem(my + t + 3, N_DEV),
        ]
        rdmas = []
        for d, (nbr, co, ho, oc, mlc, os_, or_, mls, mlr) in enumerate(dirs):
            ss = pl.ds(c_send[d] * CH, CH)
            oc[snd] = acc_o[ss, pl.ds(co, HCOL)]
            mlc[snd] = jnp.concatenate(
                [acc_ml[ss, ho:ho + HH], acc_ml[ss, HQ + ho:HQ + ho + HH]], axis=1
            )
            o_rdma = pltpu.make_async_remote_copy(
                src_ref=oc.at[snd], dst_ref=oc.at[rcv],
                send_sem=os_.at[snd], recv_sem=or_.at[rcv],
                device_id=(nbr,), device_id_type=pl.DeviceIdType.MESH,
            )
            ml_rdma = pltpu.make_async_remote_copy(
                src_ref=mlc.at[snd], dst_ref=mlc.at[rcv],
                send_sem=mls.at[snd], recv_sem=mlr.at[rcv],
                device_id=(nbr,), device_id_type=pl.DeviceIdType.MESH,
            )
            o_rdma.start()
            ml_rdma.start()
            rdmas.append((o_rdma, ml_rdma))
        for d, (nbr, co, ho, oc, mlc, *_s) in enumerate(dirs):
            rdmas[d][0].wait()
            rdmas[d][1].wait()
            rs = pl.ds(c_recv[d] * CH, CH)
            m_acc = acc_ml[rs, ho:ho + HH]
            l_acc = acc_ml[rs, HQ + ho:HQ + ho + HH]
            m_in = mlc[rcv, :, 0:HH]
            l_in = mlc[rcv, :, HH:2 * HH]
            m_new = jnp.maximum(m_acc, m_in)
            a = jnp.exp(m_acc - m_new)
            b = jnp.exp(m_in - m_new)
            acc_ml[rs, HQ + ho:HQ + ho + HH] = l_acc * a + l_in * b
            acc_ml[rs, ho:ho + HH] = m_new
            for k in range(HH):
                hs = pl.ds(co + k * DH, DH)
                acc_o[rs, hs] = (
                    acc_o[rs, hs] * a[:, k:k + 1]
                    + oc[rcv, :, pl.ds(k * DH, DH)] * b[:, k:k + 1]
                )

    q = lax.rem(my + 1, N_DEV)
    qs = pl.ds(q * CH, CH)
    ctx = jnp.concatenate(
        [
            acc_o[qs, pl.ds(hh * DH, DH)] / acc_ml[qs, HQ + hh:HQ + hh + 1]
            for hh in range(HQ)
        ],
        axis=1,
    )
    mine = jnp.dot(ctx, wo_ref[...], preferred_element_type=jnp.float32)
    _store_chunk_unpermuted(out_ref, mine, q)
    ag_cw[0] = mine[:, 0:HCOL]
    ag_ccw[0] = mine[:, HCOL:DMODEL]

    for h in range(N_DEV - 1):
        snd = h % 2
        rcv = (h + 1) % 2
        cw_rdma = pltpu.make_async_remote_copy(
            src_ref=ag_cw.at[snd], dst_ref=ag_cw.at[rcv],
            send_sem=ag_cw_s.at[snd], recv_sem=ag_cw_r.at[rcv],
            device_id=(right,), device_id_type=pl.DeviceIdType.MESH,
        )
        ccw_rdma = pltpu.make_async_remote_copy(
            src_ref=ag_ccw.at[snd], dst_ref=ag_ccw.at[rcv],
            send_sem=ag_ccw_s.at[snd], recv_sem=ag_ccw_r.at[rcv],
            device_id=(left,), device_id_type=pl.DeviceIdType.MESH,
        )
        cw_rdma.start()
        ccw_rdma.start()
        cw_rdma.wait()
        ccw_rdma.wait()
        _store_half_chunk(out_ref, ag_cw[rcv], lax.rem(my - h + N_DEV, N_DEV), 0)
        _store_half_chunk(out_ref, ag_ccw[rcv], lax.rem(my + h + 2, N_DEV), HCOL)


def kernel(x, Wq, K_ext, V_ext, Wo):
    perm = jnp.asarray(PERM, dtype=jnp.int32)
    x_p = x.reshape(SQ // BLK, BLK, DMODEL)[perm].reshape(SQ, DMODEL)
    k_p = K_ext.reshape(SKV_LOC // BLK, BLK, HQ * DH)[perm].reshape(SKV_LOC, HQ * DH)
    v_p = V_ext.reshape(SKV_LOC // BLK, BLK, HQ * DH)[perm].reshape(SKV_LOC, HQ * DH)

    o, ml = pl.pallas_call(
        _attn_body,
        in_specs=[pl.BlockSpec(memory_space=pltpu.VMEM)] * 4,
        out_shape=[
            jax.ShapeDtypeStruct((SQ, DMODEL), jnp.float32),
            jax.ShapeDtypeStruct((SQ, 2 * HQ), jnp.float32),
        ],
        out_specs=[pl.BlockSpec(memory_space=pltpu.VMEM)] * 2,
        compiler_params=pltpu.CompilerParams(vmem_limit_bytes=100 * 1024 * 1024),
    )(x_p, Wq, k_p, v_p)

    out = pl.pallas_call(
        _combine_body,
        in_specs=[pl.BlockSpec(memory_space=pltpu.VMEM)] * 3,
        out_shape=jax.ShapeDtypeStruct((SQ, DMODEL), jnp.float32),
        out_specs=pl.BlockSpec(memory_space=pltpu.VMEM),
        scratch_shapes=[
            pltpu.VMEM((SQ, DMODEL), jnp.float32),
            pltpu.VMEM((SQ, 2 * HQ), jnp.float32),
            pltpu.VMEM((2, CH, HCOL), jnp.float32),
            pltpu.VMEM((2, CH, HCOL), jnp.float32),
            pltpu.VMEM((2, CH, HQ), jnp.float32),
            pltpu.VMEM((2, CH, HQ), jnp.float32),
            pltpu.VMEM((2, CH, HCOL), jnp.float32),
            pltpu.VMEM((2, CH, HCOL), jnp.float32),
        ] + [pltpu.SemaphoreType.DMA((2,))] * 12,
        compiler_params=pltpu.CompilerParams(
            collective_id=0, vmem_limit_bytes=100 * 1024 * 1024
        ),
    )(o, ml, Wo)

    return out.reshape(1, SQ, DMODEL)


# device time: 155870 ns/iter; 5.3932x vs baseline; 1.2492x over previous
import jax
import jax.numpy as jnp
from jax import lax
from jax.experimental import pallas as pl
from jax.experimental.pallas import tpu as pltpu

N_DEV = 8
SQ = 2048
DMODEL = 1024
HQ = 8
DH = 128
SKV_LOC = 2048
BLK = 64
NRES = 4
NJ = SQ // (BLK * NRES)
RROWS = SQ // NRES
CH = SQ // N_DEV
SCALE = 0.08838834764831843

PERM = [r + NRES * j for r in range(NRES) for j in range(NJ)]


def _attn_body(x_ref, wq_ref, k_ref, v_ref, o_ref, ml_ref):
    q_all = jnp.dot(x_ref[...], wq_ref[...], preferred_element_type=jnp.float32)
    for hh in range(HQ):
        hs = pl.ds(hh * DH, DH)
        for r in range(NRES):
            rs = pl.ds(r * RROWS, RROWS)
            qr = q_all[r * RROWS:(r + 1) * RROWS, hh * DH:(hh + 1) * DH]
            kr = k_ref[rs, hs]
            vr = v_ref[rs, hs]
            s = lax.dot_general(
                qr, kr, (((1,), (1,)), ((), ())), preferred_element_type=jnp.float32
            ) * SCALE
            mr = jnp.max(s, axis=1)
            p = jnp.exp(s - mr[:, None])
            lr = jnp.sum(p, axis=1)
            orr = jnp.dot(p, vr, preferred_element_type=jnp.float32)
            o_ref[rs, hs] = orr
            ml_ref[rs, hh:hh + 1] = mr[:, None]
            ml_ref[rs, HQ + hh:HQ + hh + 1] = lr[:, None]


def _store_chunk_unpermuted(out_ref, chunk, c):
    for i in range(CH // BLK):
        p = c * (CH // BLK) + i
        blk = lax.rem(p, NJ) * NRES + lax.div(p, NJ)
        out_ref[pl.ds(blk * BLK, BLK), :] = chunk[i * BLK:(i + 1) * BLK, :]


def _store_half_chunk(out_ref, chunk, c, co):
    for i in range(CH // BLK):
        p = c * (CH // BLK) + i
        blk = lax.rem(p, NJ) * NRES + lax.div(p, NJ)
        out_ref[pl.ds(blk * BLK, BLK), pl.ds(co, HCOL)] = (
            chunk[i * BLK:(i + 1) * BLK, :].astype(jnp.float32)
        )


HCOL = DMODEL // 2
HH = HQ // 2


def _combine_body(
    o_ref, ml_ref, wo_ref, out_ref,
    acc_o, acc_ml,
    o_cw, o_ccw, ml_cw, ml_ccw, ag_cw, ag_ccw,
    o_cw_s, o_cw_r, o_ccw_s, o_ccw_r,
    ml_cw_s, ml_cw_r, ml_ccw_s, ml_ccw_r,
    ag_cw_s, ag_cw_r, ag_ccw_s, ag_ccw_r,
):
    my = lax.axis_index("i")
    left = lax.rem(my - 1 + N_DEV, N_DEV)
    right = lax.rem(my + 1, N_DEV)

    barrier = pltpu.get_barrier_semaphore()
    for nbr in (left, right):
        pl.semaphore_signal(
            barrier, inc=1, device_id=(nbr,), device_id_type=pl.DeviceIdType.MESH
        )
    pl.semaphore_wait(barrier, 2)

    acc_o[...] = o_ref[...]
    acc_ml[...] = ml_ref[...]

    dirs = [
        (right, 0, 0, o_cw, ml_cw, o_cw_s, o_cw_r, ml_cw_s, ml_cw_r),
        (left, HCOL, HH, o_ccw, ml_ccw, o_ccw_s, o_ccw_r, ml_ccw_s, ml_ccw_r),
    ]
    for t in range(N_DEV - 1):
        snd = t % 2
        rcv = (t + 1) % 2
        c_send = [
            lax.rem(my - t + N_DEV, N_DEV),
            lax.rem(my + t + 2, N_DEV),
        ]
        c_recv = [
            lax.rem(my - t - 1 + 2 * N_DEV, N_DEV),
            lax.rem(my + t + 3, N_DEV),
        ]
        rdmas = []
        for d, (nbr, co, ho, oc, mlc, os_, or_, mls, mlr) in enumerate(dirs):
            ss = pl.ds(c_send[d] * CH, CH)
            oc[snd] = acc_o[ss, pl.ds(co, HCOL)].astype(jnp.bfloat16)
            mlc[snd] = jnp.concatenate(
                [acc_ml[ss, ho:ho + HH], acc_ml[ss, HQ + ho:HQ + ho + HH]], axis=1
            )
            o_rdma = pltpu.make_async_remote_copy(
                src_ref=oc.at[snd], dst_ref=oc.at[rcv],
                send_sem=os_.at[snd], recv_sem=or_.at[rcv],
                device_id=(nbr,), device_id_type=pl.DeviceIdType.MESH,
            )
            ml_rdma = pltpu.make_async_remote_copy(
                src_ref=mlc.at[snd], dst_ref=mlc.at[rcv],
                send_sem=mls.at[snd], recv_sem=mlr.at[rcv],
                device_id=(nbr,), device_id_type=pl.DeviceIdType.MESH,
            )
            o_rdma.start()
            ml_rdma.start()
            rdmas.append((o_rdma, ml_rdma))
        for d, (nbr, co, ho, oc, mlc, *_s) in enumerate(dirs):
            rdmas[d][0].wait()
            rdmas[d][1].wait()
            rs = pl.ds(c_recv[d] * CH, CH)
            m_acc = acc_ml[rs, ho:ho + HH]
            l_acc = acc_ml[rs, HQ + ho:HQ + ho + HH]
            m_in = mlc[rcv, :, 0:HH]
            l_in = mlc[rcv, :, HH:2 * HH]
            m_new = jnp.maximum(m_acc, m_in)
            a = jnp.exp(m_acc - m_new)
            b = jnp.exp(m_in - m_new)
            acc_ml[rs, HQ + ho:HQ + ho + HH] = l_acc * a + l_in * b
            acc_ml[rs, ho:ho + HH] = m_new
            for k in range(HH):
                hs = pl.ds(co + k * DH, DH)
                acc_o[rs, hs] = (
                    acc_o[rs, hs] * a[:, k:k + 1]
                    + oc[rcv, :, pl.ds(k * DH, DH)].astype(jnp.float32)
                    * b[:, k:k + 1]
                )

    q = lax.rem(my + 1, N_DEV)
    qs = pl.ds(q * CH, CH)
    ctx = jnp.concatenate(
        [
            acc_o[qs, pl.ds(hh * DH, DH)] / acc_ml[qs, HQ + hh:HQ + hh + 1]
            for hh in range(HQ)
        ],
        axis=1,
    )
    mine = jnp.dot(ctx, wo_ref[...], preferred_element_type=jnp.float32)
    _store_chunk_unpermuted(out_ref, mine, q)
    ag_cw[0] = mine[:, 0:HCOL].astype(jnp.bfloat16)
    ag_ccw[0] = mine[:, HCOL:DMODEL].astype(jnp.bfloat16)

    for h in range(N_DEV - 1):
        snd = h % 2
        rcv = (h + 1) % 2
        cw_rdma = pltpu.make_async_remote_copy(
            src_ref=ag_cw.at[snd], dst_ref=ag_cw.at[rcv],
            send_sem=ag_cw_s.at[snd], recv_sem=ag_cw_r.at[rcv],
            device_id=(right,), device_id_type=pl.DeviceIdType.MESH,
        )
        ccw_rdma = pltpu.make_async_remote_copy(
            src_ref=ag_ccw.at[snd], dst_ref=ag_ccw.at[rcv],
            send_sem=ag_ccw_s.at[snd], recv_sem=ag_ccw_r.at[rcv],
            device_id=(left,), device_id_type=pl.DeviceIdType.MESH,
        )
        cw_rdma.start()
        ccw_rdma.start()
        cw_rdma.wait()
        ccw_rdma.wait()
        _store_half_chunk(out_ref, ag_cw[rcv], lax.rem(my - h + N_DEV, N_DEV), 0)
        _store_half_chunk(out_ref, ag_ccw[rcv], lax.rem(my + h + 2, N_DEV), HCOL)


def kernel(x, Wq, K_ext, V_ext, Wo):
    perm = jnp.asarray(PERM, dtype=jnp.int32)
    x_p = x.reshape(SQ // BLK, BLK, DMODEL)[perm].reshape(SQ, DMODEL)
    k_p = K_ext.reshape(SKV_LOC // BLK, BLK, HQ * DH)[perm].reshape(SKV_LOC, HQ * DH)
    v_p = V_ext.reshape(SKV_LOC // BLK, BLK, HQ * DH)[perm].reshape(SKV_LOC, HQ * DH)

    o, ml = pl.pallas_call(
        _attn_body,
        in_specs=[pl.BlockSpec(memory_space=pltpu.VMEM)] * 4,
        out_shape=[
            jax.ShapeDtypeStruct((SQ, DMODEL), jnp.float32),
            jax.ShapeDtypeStruct((SQ, 2 * HQ), jnp.float32),
        ],
        out_specs=[pl.BlockSpec(memory_space=pltpu.VMEM)] * 2,
        compiler_params=pltpu.CompilerParams(vmem_limit_bytes=100 * 1024 * 1024),
    )(x_p, Wq, k_p, v_p)

    out = pl.pallas_call(
        _combine_body,
        in_specs=[pl.BlockSpec(memory_space=pltpu.VMEM)] * 3,
        out_shape=jax.ShapeDtypeStruct((SQ, DMODEL), jnp.float32),
        out_specs=pl.BlockSpec(memory_space=pltpu.VMEM),
        scratch_shapes=[
            pltpu.VMEM((SQ, DMODEL), jnp.float32),
            pltpu.VMEM((SQ, 2 * HQ), jnp.float32),
            pltpu.VMEM((2, CH, HCOL), jnp.bfloat16),
            pltpu.VMEM((2, CH, HCOL), jnp.bfloat16),
            pltpu.VMEM((2, CH, HQ), jnp.float32),
            pltpu.VMEM((2, CH, HQ), jnp.float32),
            pltpu.VMEM((2, CH, HCOL), jnp.bfloat16),
            pltpu.VMEM((2, CH, HCOL), jnp.bfloat16),
        ] + [pltpu.SemaphoreType.DMA((2,))] * 12,
        compiler_params=pltpu.CompilerParams(
            collective_id=0, vmem_limit_bytes=100 * 1024 * 1024
        ),
    )(o, ml, Wo)

    return out.reshape(1, SQ, DMODEL)


# device time: 155463 ns/iter; 5.4074x vs baseline; 1.0026x over previous
import jax
import jax.numpy as jnp
from jax import lax
from jax.experimental import pallas as pl
from jax.experimental.pallas import tpu as pltpu

N_DEV = 8
SQ = 2048
DMODEL = 1024
HQ = 8
DH = 128
SKV_LOC = 2048
BLK = 64
NRES = 4
NJ = SQ // (BLK * NRES)
RROWS = SQ // NRES
CH = SQ // N_DEV
SCALE = 0.08838834764831843

PERM = [r + NRES * j for r in range(NRES) for j in range(NJ)]


def _attn_body(x_v, wq_ref, k_v, v_v, o_ref, ml_ref):
    q_bf = jnp.dot(
        x_v[...].astype(jnp.bfloat16),
        wq_ref[...].astype(jnp.bfloat16),
        preferred_element_type=jnp.float32,
    ).astype(jnp.bfloat16)
    for hh in range(HQ):
        hs = pl.ds(hh * DH, DH)
        for r in range(NRES):
            rs = pl.ds(r * RROWS, RROWS)
            qr = q_bf[r * RROWS:(r + 1) * RROWS, hh * DH:(hh + 1) * DH]
            kr = k_v[rs, hs]
            vr = v_v[rs, hs]
            s = lax.dot_general(
                qr,
                kr.astype(jnp.bfloat16),
                (((1,), (1,)), ((), ())),
                preferred_element_type=jnp.float32,
            ) * SCALE
            mr = jnp.max(s, axis=1)
            p = jnp.exp(s - mr[:, None])
            lr = jnp.sum(p, axis=1)
            orr = jnp.dot(
                p.astype(jnp.bfloat16),
                vr.astype(jnp.bfloat16),
                preferred_element_type=jnp.float32,
            )
            o_ref[rs, hs] = orr.astype(jnp.bfloat16)
            ml_ref[rs, hh:hh + 1] = mr[:, None]
            ml_ref[rs, HQ + hh:HQ + hh + 1] = lr[:, None]


def _store_chunk_unpermuted(out_ref, chunk, c):
    for i in range(CH // BLK):
        p = c * (CH // BLK) + i
        blk = lax.rem(p, NJ) * NRES + lax.div(p, NJ)
        out_ref[pl.ds(blk * BLK, BLK), :] = chunk[i * BLK:(i + 1) * BLK, :]


def _store_half_chunk(out_ref, chunk, c, co):
    for i in range(CH // BLK):
        p = c * (CH // BLK) + i
        blk = lax.rem(p, NJ) * NRES + lax.div(p, NJ)
        out_ref[pl.ds(blk * BLK, BLK), pl.ds(co, HCOL)] = (
            chunk[i * BLK:(i + 1) * BLK, :].astype(jnp.float32)
        )


HCOL = DMODEL // 2
HH = HQ // 2


def _combine_body(
    o_ref, ml_ref, wo_ref, out_ref,
    acc_o, acc_ml,
    o_cw, o_ccw, ml_cw, ml_ccw, ag_cw, ag_ccw,
    o_cw_s, o_cw_r, o_ccw_s, o_ccw_r,
    ml_cw_s, ml_cw_r, ml_ccw_s, ml_ccw_r,
    ag_cw_s, ag_cw_r, ag_ccw_s, ag_ccw_r,
):
    my = lax.axis_index("i")
    left = lax.rem(my - 1 + N_DEV, N_DEV)
    right = lax.rem(my + 1, N_DEV)

    barrier = pltpu.get_barrier_semaphore()
    for nbr in (left, right):
        pl.semaphore_signal(
            barrier, inc=1, device_id=(nbr,), device_id_type=pl.DeviceIdType.MESH
        )
    pl.semaphore_wait(barrier, 2)

    acc_o[...] = o_ref[...].astype(jnp.float32)
    acc_ml[...] = ml_ref[...]

    dirs = [
        (right, 0, 0, o_cw, ml_cw, o_cw_s, o_cw_r, ml_cw_s, ml_cw_r),
        (left, HCOL, HH, o_ccw, ml_ccw, o_ccw_s, o_ccw_r, ml_ccw_s, ml_ccw_r),
    ]
    for t in range(N_DEV - 1):
        snd = t % 2
        rcv = (t + 1) % 2
        c_send = [
            lax.rem(my - t + N_DEV, N_DEV),
            lax.rem(my + t + 2, N_DEV),
        ]
        c_recv = [
            lax.rem(my - t - 1 + 2 * N_DEV, N_DEV),
            lax.rem(my + t + 3, N_DEV),
        ]
        rdmas = []
        for d, (nbr, co, ho, oc, mlc, os_, or_, mls, mlr) in enumerate(dirs):
            ss = pl.ds(c_send[d] * CH, CH)
            oc[snd] = acc_o[ss, pl.ds(co, HCOL)].astype(jnp.bfloat16)
            mlc[snd] = jnp.concatenate(
                [acc_ml[ss, ho:ho + HH], acc_ml[ss, HQ + ho:HQ + ho + HH]], axis=1
            )
            o_rdma = pltpu.make_async_remote_copy(
                src_ref=oc.at[snd], dst_ref=oc.at[rcv],
                send_sem=os_.at[snd], recv_sem=or_.at[rcv],
                device_id=(nbr,), device_id_type=pl.DeviceIdType.MESH,
            )
            ml_rdma = pltpu.make_async_remote_copy(
                src_ref=mlc.at[snd], dst_ref=mlc.at[rcv],
                send_sem=mls.at[snd], recv_sem=mlr.at[rcv],
                device_id=(nbr,), device_id_type=pl.DeviceIdType.MESH,
            )
            o_rdma.start()
            ml_rdma.start()
            rdmas.append((o_rdma, ml_rdma))
        for d, (nbr, co, ho, oc, mlc, *_s) in enumerate(dirs):
            rdmas[d][0].wait()
            rdmas[d][1].wait()
            rs = pl.ds(c_recv[d] * CH, CH)
            m_acc = acc_ml[rs, ho:ho + HH]
            l_acc = acc_ml[rs, HQ + ho:HQ + ho + HH]
            m_in = mlc[rcv, :, 0:HH]
            l_in = mlc[rcv, :, HH:2 * HH]
            m_new = jnp.maximum(m_acc, m_in)
            a = jnp.exp(m_acc - m_new)
            b = jnp.exp(m_in - m_new)
            acc_ml[rs, HQ + ho:HQ + ho + HH] = l_acc * a + l_in * b
            acc_ml[rs, ho:ho + HH] = m_new
            for k in range(HH):
                hs = pl.ds(co + k * DH, DH)
                acc_o[rs, hs] = (
                    acc_o[rs, hs] * a[:, k:k + 1]
                    + oc[rcv, :, pl.ds(k * DH, DH)].astype(jnp.float32)
                    * b[:, k:k + 1]
                )

    q = lax.rem(my + 1, N_DEV)
    qs = pl.ds(q * CH, CH)
    ctx = jnp.concatenate(
        [
            acc_o[qs, pl.ds(hh * DH, DH)] / acc_ml[qs, HQ + hh:HQ + hh + 1]
            for hh in range(HQ)
        ],
        axis=1,
    )
    mine = jnp.dot(ctx, wo_ref[...], preferred_element_type=jnp.float32)
    _store_chunk_unpermuted(out_ref, mine, q)
    ag_cw[0] = mine[:, 0:HCOL].astype(jnp.bfloat16)
    ag_ccw[0] = mine[:, HCOL:DMODEL].astype(jnp.bfloat16)

    for h in range(N_DEV - 1):
        snd = h % 2
        rcv = (h + 1) % 2
        cw_rdma = pltpu.make_async_remote_copy(
            src_ref=ag_cw.at[snd], dst_ref=ag_cw.at[rcv],
            send_sem=ag_cw_s.at[snd], recv_sem=ag_cw_r.at[rcv],
            device_id=(right,), device_id_type=pl.DeviceIdType.MESH,
        )
        ccw_rdma = pltpu.make_async_remote_copy(
            src_ref=ag_ccw.at[snd], dst_ref=ag_ccw.at[rcv],
            send_sem=ag_ccw_s.at[snd], recv_sem=ag_ccw_r.at[rcv],
            device_id=(left,), device_id_type=pl.DeviceIdType.MESH,
        )
        cw_rdma.start()
        ccw_rdma.start()
        cw_rdma.wait()
        ccw_rdma.wait()
        _store_half_chunk(out_ref, ag_cw[rcv], lax.rem(my - h + N_DEV, N_DEV), 0)
        _store_half_chunk(out_ref, ag_ccw[rcv], lax.rem(my + h + 2, N_DEV), HCOL)


def kernel(x, Wq, K_ext, V_ext, Wo):
    perm = jnp.asarray(PERM, dtype=jnp.int32)
    x_p = x.reshape(SQ // BLK, BLK, DMODEL)[perm].reshape(SQ, DMODEL)
    k_p = K_ext.reshape(SKV_LOC // BLK, BLK, HQ * DH)[perm].reshape(SKV_LOC, HQ * DH)
    v_p = V_ext.reshape(SKV_LOC // BLK, BLK, HQ * DH)[perm].reshape(SKV_LOC, HQ * DH)

    o, ml = pl.pallas_call(
        _attn_body,
        in_specs=[pl.BlockSpec(memory_space=pltpu.VMEM)] * 4,
        out_shape=[
            jax.ShapeDtypeStruct((SQ, DMODEL), jnp.bfloat16),
            jax.ShapeDtypeStruct((SQ, 2 * HQ), jnp.float32),
        ],
        out_specs=[pl.BlockSpec(memory_space=pltpu.VMEM)] * 2,
        compiler_params=pltpu.CompilerParams(vmem_limit_bytes=100 * 1024 * 1024),
    )(x_p, Wq, k_p, v_p)

    out = pl.pallas_call(
        _combine_body,
        in_specs=[pl.BlockSpec(memory_space=pltpu.VMEM)] * 3,
        out_shape=jax.ShapeDtypeStruct((SQ, DMODEL), jnp.float32),
        out_specs=pl.BlockSpec(memory_space=pltpu.VMEM),
        scratch_shapes=[
            pltpu.VMEM((SQ, DMODEL), jnp.float32),
            pltpu.VMEM((SQ, 2 * HQ), jnp.float32),
            pltpu.VMEM((2, CH, HCOL), jnp.bfloat16),
            pltpu.VMEM((2, CH, HCOL), jnp.bfloat16),
            pltpu.VMEM((2, CH, HQ), jnp.float32),
            pltpu.VMEM((2, CH, HQ), jnp.float32),
            pltpu.VMEM((2, CH, HCOL), jnp.bfloat16),
            pltpu.VMEM((2, CH, HCOL), jnp.bfloat16),
        ] + [pltpu.SemaphoreType.DMA((2,))] * 12,
        compiler_params=pltpu.CompilerParams(
            collective_id=0, vmem_limit_bytes=100 * 1024 * 1024
        ),
    )(o, ml, Wo)

    return out.reshape(1, SQ, DMODEL)


# device time: 138498 ns/iter; 6.0697x vs baseline; 1.1225x over previous
import jax
import jax.numpy as jnp
from jax import lax
from jax.experimental import pallas as pl
from jax.experimental.pallas import tpu as pltpu

N_DEV = 8
SQ = 2048
DMODEL = 1024
HQ = 8
DH = 128
SKV_LOC = 2048
BLK = 64
NRES = 4
NJ = SQ // (BLK * NRES)
RROWS = SQ // NRES
CH = SQ // N_DEV
SCALE = 0.08838834764831843

PERM = [r + NRES * j for r in range(NRES) for j in range(NJ)]


def _attn_body(x_v, wq_ref, k_v, v_v, o_ref, ml_ref):
    q_bf = jnp.dot(
        x_v[...].astype(jnp.bfloat16),
        wq_ref[...].astype(jnp.bfloat16),
        preferred_element_type=jnp.float32,
    ).astype(jnp.bfloat16)
    for hh in range(HQ):
        hs = pl.ds(hh * DH, DH)
        for r in range(NRES):
            rs = pl.ds(r * RROWS, RROWS)
            qr = q_bf[r * RROWS:(r + 1) * RROWS, hh * DH:(hh + 1) * DH]
            kr = k_v[rs, hs]
            vr = v_v[rs, hs]
            s = lax.dot_general(
                qr,
                kr.astype(jnp.bfloat16),
                (((1,), (1,)), ((), ())),
                preferred_element_type=jnp.float32,
            ) * SCALE
            mr = jnp.max(s, axis=1)
            p = jnp.exp(s - mr[:, None])
            lr = jnp.sum(p, axis=1)
            orr = jnp.dot(
                p.astype(jnp.bfloat16),
                vr.astype(jnp.bfloat16),
                preferred_element_type=jnp.float32,
            )
            o_ref[rs, hs] = orr.astype(jnp.bfloat16)
            ml_ref[rs, hh:hh + 1] = mr[:, None]
            ml_ref[rs, HQ + hh:HQ + hh + 1] = lr[:, None]


def _store_chunk_unpermuted(out_ref, chunk, c):
    for i in range(CH // BLK):
        p = c * (CH // BLK) + i
        blk = lax.rem(p, NJ) * NRES + lax.div(p, NJ)
        out_ref[pl.ds(blk * BLK, BLK), :] = chunk[i * BLK:(i + 1) * BLK, :]


def _store_half_chunk(out_ref, chunk, c, co):
    for i in range(CH // BLK):
        p = c * (CH // BLK) + i
        blk = lax.rem(p, NJ) * NRES + lax.div(p, NJ)
        out_ref[pl.ds(blk * BLK, BLK), pl.ds(co, HCOL)] = (
            chunk[i * BLK:(i + 1) * BLK, :].astype(jnp.float32)
        )


HCOL = DMODEL // 2
HH = HQ // 2

RS_MASKS = [[1, 3, 4], [4, 1, 3]]
RS_SPANS = [
    [[0, 3, 4, 7], [0, 4], [0]],
    [[0, 1, 3, 2], [0, 3], [0]],
]
RS_OFF = [0, 4 * CH, 6 * CH]
AG_MASKS = [[4, 3, 1], [3, 1, 4]]
AG_G = [[0, 4, 3, 7, 1, 5, 2, 6], [0, 3, 1, 2, 4, 7, 5, 6]]


def _combine_body(
    o_ref, ml_ref, wo_ref, out_ref,
    acc_o, acc_ml,
    o_snd0, o_rcv0, ml_snd0, ml_rcv0, ag0,
    o_snd1, o_rcv1, ml_snd1, ml_rcv1, ag1,
    o_ssem0, o_rsem0, ml_ssem0, ml_rsem0, ag_ssem0, ag_rsem0,
    o_ssem1, o_rsem1, ml_ssem1, ml_rsem1, ag_ssem1, ag_rsem1,
):
    my = lax.axis_index("i")
    trees = [
        (0, 0, o_snd0, o_rcv0, ml_snd0, ml_rcv0, ag0,
         o_ssem0, o_rsem0, ml_ssem0, ml_rsem0, ag_ssem0, ag_rsem0),
        (HCOL, HH, o_snd1, o_rcv1, ml_snd1, ml_rcv1, ag1,
         o_ssem1, o_rsem1, ml_ssem1, ml_rsem1, ag_ssem1, ag_rsem1),
    ]

    barrier = pltpu.get_barrier_semaphore()
    for mask in (1, 3, 4):
        pl.semaphore_signal(
            barrier, inc=1, device_id=(my ^ mask,),
            device_id_type=pl.DeviceIdType.MESH,
        )
    pl.semaphore_wait(barrier, 3)

    acc_o[...] = o_ref[...].astype(jnp.float32)
    acc_ml[...] = ml_ref[...]

    for k in range(3):
        off = RS_OFF[k]
        rdmas = []
        for d, (co, ho, o_snd, o_rcv, ml_snd, ml_rcv, _ag,
                o_ss, o_rs, ml_ss, ml_rs, _as, _ar) in enumerate(trees):
            mask = RS_MASKS[d][k]
            span = RS_SPANS[d][k]
            n = len(span)
            for idx, s in enumerate(span):
                src = pl.ds((my ^ (mask ^ s)) * CH, CH)
                dst = pl.ds(off + idx * CH, CH)
                o_snd[dst, :] = acc_o[src, pl.ds(co, HCOL)].astype(jnp.bfloat16)
                ml_snd[dst, :] = jnp.concatenate(
                    [acc_ml[src, ho:ho + HH], acc_ml[src, HQ + ho:HQ + ho + HH]],
                    axis=1,
                )
            o_rdma = pltpu.make_async_remote_copy(
                src_ref=o_snd.at[pl.ds(off, n * CH), :],
                dst_ref=o_rcv.at[pl.ds(off, n * CH), :],
                send_sem=o_ss.at[k], recv_sem=o_rs.at[k],
                device_id=(my ^ mask,), device_id_type=pl.DeviceIdType.MESH,
            )
            ml_rdma = pltpu.make_async_remote_copy(
                src_ref=ml_snd.at[pl.ds(off, n * CH), :],
                dst_ref=ml_rcv.at[pl.ds(off, n * CH), :],
                send_sem=ml_ss.at[k], recv_sem=ml_rs.at[k],
                device_id=(my ^ mask,), device_id_type=pl.DeviceIdType.MESH,
            )
            o_rdma.start()
            ml_rdma.start()
            rdmas.append((o_rdma, ml_rdma))
        for d, (co, ho, o_snd, o_rcv, ml_snd, ml_rcv, *_r) in enumerate(trees):
            rdmas[d][0].wait()
            rdmas[d][1].wait()
            span = RS_SPANS[d][k]
            for idx, s in enumerate(span):
                rs = pl.ds((my ^ s) * CH, CH)
                rr = pl.ds(off + idx * CH, CH)
                m_acc = acc_ml[rs, ho:ho + HH]
                l_acc = acc_ml[rs, HQ + ho:HQ + ho + HH]
                m_in = ml_rcv[rr, 0:HH]
                l_in = ml_rcv[rr, HH:2 * HH]
                m_new = jnp.maximum(m_acc, m_in)
                a = jnp.exp(m_acc - m_new)
                b = jnp.exp(m_in - m_new)
                acc_ml[rs, HQ + ho:HQ + ho + HH] = l_acc * a + l_in * b
                acc_ml[rs, ho:ho + HH] = m_new
                for h in range(HH):
                    hs = pl.ds(co + h * DH, DH)
                    acc_o[rs, hs] = (
                        acc_o[rs, hs] * a[:, h:h + 1]
                        + o_rcv[rr, pl.ds(h * DH, DH)].astype(jnp.float32)
                        * b[:, h:h + 1]
                    )

    qs = pl.ds(my * CH, CH)
    ctx = jnp.concatenate(
        [
            acc_o[qs, pl.ds(hh * DH, DH)] / acc_ml[qs, HQ + hh:HQ + hh + 1]
            for hh in range(HQ)
        ],
        axis=1,
    )
    mine = jnp.dot(ctx, wo_ref[...], preferred_element_type=jnp.float32)
    _store_chunk_unpermuted(out_ref, mine, my)
    ag0[pl.ds(0, CH), :] = mine[:, 0:HCOL].astype(jnp.bfloat16)
    ag1[pl.ds(0, CH), :] = mine[:, HCOL:DMODEL].astype(jnp.bfloat16)

    for k in range(3):
        n = 1 << k
        rdmas = []
        for d, (co, ho, *_b) in enumerate(trees):
            ag = trees[d][6]
            ag_ss = trees[d][11]
            ag_rs = trees[d][12]
            mask = AG_MASKS[d][k]
            rdma = pltpu.make_async_remote_copy(
                src_ref=ag.at[pl.ds(0, n * CH), :],
                dst_ref=ag.at[pl.ds(n * CH, n * CH), :],
                send_sem=ag_ss.at[k], recv_sem=ag_rs.at[k],
                device_id=(my ^ mask,), device_id_type=pl.DeviceIdType.MESH,
            )
            rdma.start()
            rdmas.append(rdma)
        for d in range(2):
            rdmas[d].wait()
            ag = trees[d][6]
            co = trees[d][0]
            for j in range(n, 2 * n):
                _store_half_chunk(
                    out_ref, ag[pl.ds(j * CH, CH), :], my ^ AG_G[d][j], co
                )


def kernel(x, Wq, K_ext, V_ext, Wo):
    perm = jnp.asarray(PERM, dtype=jnp.int32)
    x_p = x.reshape(SQ // BLK, BLK, DMODEL)[perm].reshape(SQ, DMODEL)
    k_p = K_ext.reshape(SKV_LOC // BLK, BLK, HQ * DH)[perm].reshape(SKV_LOC, HQ * DH)
    v_p = V_ext.reshape(SKV_LOC // BLK, BLK, HQ * DH)[perm].reshape(SKV_LOC, HQ * DH)

    o, ml = pl.pallas_call(
        _attn_body,
        in_specs=[pl.BlockSpec(memory_space=pltpu.VMEM)] * 4,
        out_shape=[
            jax.ShapeDtypeStruct((SQ, DMODEL), jnp.bfloat16),
            jax.ShapeDtypeStruct((SQ, 2 * HQ), jnp.float32),
        ],
        out_specs=[pl.BlockSpec(memory_space=pltpu.VMEM)] * 2,
        compiler_params=pltpu.CompilerParams(vmem_limit_bytes=100 * 1024 * 1024),
    )(x_p, Wq, k_p, v_p)

    out = pl.pallas_call(
        _combine_body,
        in_specs=[pl.BlockSpec(memory_space=pltpu.VMEM)] * 3,
        out_shape=jax.ShapeDtypeStruct((SQ, DMODEL), jnp.float32),
        out_specs=pl.BlockSpec(memory_space=pltpu.VMEM),
        scratch_shapes=[
            pltpu.VMEM((SQ, DMODEL), jnp.float32),
            pltpu.VMEM((SQ, 2 * HQ), jnp.float32),
        ] + [
            pltpu.VMEM((7 * CH, HCOL), jnp.bfloat16),
            pltpu.VMEM((7 * CH, HCOL), jnp.bfloat16),
            pltpu.VMEM((7 * CH, HQ), jnp.float32),
            pltpu.VMEM((7 * CH, HQ), jnp.float32),
            pltpu.VMEM((SQ, HCOL), jnp.bfloat16),
        ] * 2 + [pltpu.SemaphoreType.DMA((3,))] * 12,
        compiler_params=pltpu.CompilerParams(
            collective_id=0, vmem_limit_bytes=100 * 1024 * 1024
        ),
    )(o, ml, Wo)

    return out.reshape(1, SQ, DMODEL)


# device time: 137919 ns/iter; 6.0952x vs baseline; 1.0042x over previous
import jax
import jax.numpy as jnp
from jax import lax
from jax.experimental import pallas as pl
from jax.experimental.pallas import tpu as pltpu

N_DEV = 8
SQ = 2048
DMODEL = 1024
HQ = 8
DH = 128
SKV_LOC = 2048
BLK = 64
NRES = 4
NJ = SQ // (BLK * NRES)
RROWS = SQ // NRES
CH = SQ // N_DEV
SCALE = 0.08838834764831843

PERM = [r + NRES * j for r in range(NRES) for j in range(NJ)]


def _attn_body(x_v, wq_ref, k_v, v_v, o_ref, ml_ref):
    q_bf = jnp.dot(
        x_v[...], wq_ref[...], preferred_element_type=jnp.float32
    ).astype(jnp.bfloat16)
    for r in range(NRES):
        rs = pl.ds(r * RROWS, RROWS)
        mrs = []
        lrs = []
        for hh in range(HQ):
            hs = pl.ds(hh * DH, DH)
            qr = q_bf[r * RROWS:(r + 1) * RROWS, hh * DH:(hh + 1) * DH]
            s = lax.dot_general(
                qr, k_v[rs, hs], (((1,), (1,)), ((), ())),
                preferred_element_type=jnp.float32,
            ) * SCALE
            mr = jnp.max(s, axis=1)
            p = jnp.exp(s - mr[:, None])
            lr = jnp.sum(p, axis=1)
            orr = jnp.dot(
                p.astype(jnp.bfloat16), v_v[rs, hs],
                preferred_element_type=jnp.float32,
            )
            o_ref[rs, hs] = orr.astype(jnp.bfloat16)
            mrs.append(mr[:, None])
            lrs.append(lr[:, None])
        ml_ref[rs, 0:HQ] = jnp.concatenate(mrs, axis=1)
        ml_ref[rs, HQ:2 * HQ] = jnp.concatenate(lrs, axis=1)


def _store_chunk_unpermuted(out_ref, chunk, c):
    for i in range(CH // BLK):
        p = c * (CH // BLK) + i
        blk = lax.rem(p, NJ) * NRES + lax.div(p, NJ)
        out_ref[pl.ds(blk * BLK, BLK), :] = chunk[i * BLK:(i + 1) * BLK, :]


def _store_half_chunk(out_ref, chunk, c, co):
    for i in range(CH // BLK):
        p = c * (CH // BLK) + i
        blk = lax.rem(p, NJ) * NRES + lax.div(p, NJ)
        out_ref[pl.ds(blk * BLK, BLK), pl.ds(co, HCOL)] = (
            chunk[i * BLK:(i + 1) * BLK, :].astype(jnp.float32)
        )


HCOL = DMODEL // 2
HH = HQ // 2

RS_MASKS = [[1, 3, 4], [4, 1, 3]]
RS_SPANS = [
    [[0, 3, 4, 7], [0, 4], [0]],
    [[0, 1, 3, 2], [0, 3], [0]],
]
RS_OFF = [0, 4 * CH, 6 * CH]
AG_MASKS = [[4, 3, 1], [3, 1, 4]]
AG_G = [[0, 4, 3, 7, 1, 5, 2, 6], [0, 3, 1, 2, 4, 7, 5, 6]]


def _combine_body(
    o_ref, ml_ref, wo_ref, out_ref,
    acc_o, acc_ml,
    o_snd0, o_rcv0, ml_snd0, ml_rcv0, ag0,
    o_snd1, o_rcv1, ml_snd1, ml_rcv1, ag1,
    o_ssem0, o_rsem0, ml_ssem0, ml_rsem0, ag_ssem0, ag_rsem0,
    o_ssem1, o_rsem1, ml_ssem1, ml_rsem1, ag_ssem1, ag_rsem1,
):
    my = lax.axis_index("i")
    trees = [
        (0, 0, o_snd0, o_rcv0, ml_snd0, ml_rcv0, ag0,
         o_ssem0, o_rsem0, ml_ssem0, ml_rsem0, ag_ssem0, ag_rsem0),
        (HCOL, HH, o_snd1, o_rcv1, ml_snd1, ml_rcv1, ag1,
         o_ssem1, o_rsem1, ml_ssem1, ml_rsem1, ag_ssem1, ag_rsem1),
    ]

    barrier = pltpu.get_barrier_semaphore()
    for mask in (1, 3, 4):
        pl.semaphore_signal(
            barrier, inc=1, device_id=(my ^ mask,),
            device_id_type=pl.DeviceIdType.MESH,
        )
    pl.semaphore_wait(barrier, 3)

    acc_o[...] = o_ref[...].astype(jnp.float32)
    acc_ml[...] = ml_ref[...]

    for k in range(3):
        off = RS_OFF[k]
        rdmas = []
        for d, (co, ho, o_snd, o_rcv, ml_snd, ml_rcv, _ag,
                o_ss, o_rs, ml_ss, ml_rs, _as, _ar) in enumerate(trees):
            mask = RS_MASKS[d][k]
            span = RS_SPANS[d][k]
            n = len(span)
            for idx, s in enumerate(span):
                src = pl.ds((my ^ (mask ^ s)) * CH, CH)
                dst = pl.ds(off + idx * CH, CH)
                o_snd[dst, :] = acc_o[src, pl.ds(co, HCOL)].astype(jnp.bfloat16)
                ml_snd[dst, :] = jnp.concatenate(
                    [acc_ml[src, ho:ho + HH], acc_ml[src, HQ + ho:HQ + ho + HH]],
                    axis=1,
                )
            o_rdma = pltpu.make_async_remote_copy(
                src_ref=o_snd.at[pl.ds(off, n * CH), :],
                dst_ref=o_rcv.at[pl.ds(off, n * CH), :],
                send_sem=o_ss.at[k], recv_sem=o_rs.at[k],
                device_id=(my ^ mask,), device_id_type=pl.DeviceIdType.MESH,
            )
            ml_rdma = pltpu.make_async_remote_copy(
                src_ref=ml_snd.at[pl.ds(off, n * CH), :],
                dst_ref=ml_rcv.at[pl.ds(off, n * CH), :],
                send_sem=ml_ss.at[k], recv_sem=ml_rs.at[k],
                device_id=(my ^ mask,), device_id_type=pl.DeviceIdType.MESH,
            )
            o_rdma.start()
            ml_rdma.start()
            rdmas.append((o_rdma, ml_rdma))
        for d, (co, ho, o_snd, o_rcv, ml_snd, ml_rcv, *_r) in enumerate(trees):
            rdmas[d][0].wait()
            rdmas[d][1].wait()
            span = RS_SPANS[d][k]
            for idx, s in enumerate(span):
                rs = pl.ds((my ^ s) * CH, CH)
                rr = pl.ds(off + idx * CH, CH)
                m_acc = acc_ml[rs, ho:ho + HH]
                l_acc = acc_ml[rs, HQ + ho:HQ + ho + HH]
                m_in = ml_rcv[rr, 0:HH]
                l_in = ml_rcv[rr, HH:2 * HH]
                m_new = jnp.maximum(m_acc, m_in)
                a = jnp.exp(m_acc - m_new)
                b = jnp.exp(m_in - m_new)
                acc_ml[rs, HQ + ho:HQ + ho + HH] = l_acc * a + l_in * b
                acc_ml[rs, ho:ho + HH] = m_new
                for h in range(HH):
                    hs = pl.ds(co + h * DH, DH)
                    acc_o[rs, hs] = (
                        acc_o[rs, hs] * a[:, h:h + 1]
                        + o_rcv[rr, pl.ds(h * DH, DH)].astype(jnp.float32)
                        * b[:, h:h + 1]
                    )

    qs = pl.ds(my * CH, CH)
    ctx = jnp.concatenate(
        [
            acc_o[qs, pl.ds(hh * DH, DH)] / acc_ml[qs, HQ + hh:HQ + hh + 1]
            for hh in range(HQ)
        ],
        axis=1,
    )
    mine = jnp.dot(
        ctx.astype(jnp.bfloat16), wo_ref[...],
        preferred_element_type=jnp.float32,
    )
    _store_chunk_unpermuted(out_ref, mine, my)
    ag0[pl.ds(0, CH), :] = mine[:, 0:HCOL].astype(jnp.bfloat16)
    ag1[pl.ds(0, CH), :] = mine[:, HCOL:DMODEL].astype(jnp.bfloat16)

    for k in range(3):
        n = 1 << k
        rdmas = []
        for d, (co, ho, *_b) in enumerate(trees):
            ag = trees[d][6]
            ag_ss = trees[d][11]
            ag_rs = trees[d][12]
            mask = AG_MASKS[d][k]
            rdma = pltpu.make_async_remote_copy(
                src_ref=ag.at[pl.ds(0, n * CH), :],
                dst_ref=ag.at[pl.ds(n * CH, n * CH), :],
                send_sem=ag_ss.at[k], recv_sem=ag_rs.at[k],
                device_id=(my ^ mask,), device_id_type=pl.DeviceIdType.MESH,
            )
            rdma.start()
            rdmas.append(rdma)
        for d in range(2):
            rdmas[d].wait()
            ag = trees[d][6]
            co = trees[d][0]
            for j in range(n, 2 * n):
                _store_half_chunk(
                    out_ref, ag[pl.ds(j * CH, CH), :], my ^ AG_G[d][j], co
                )


def kernel(x, Wq, K_ext, V_ext, Wo):
    perm = jnp.asarray(PERM, dtype=jnp.int32)
    bf = jnp.bfloat16
    x_p = x.reshape(SQ // BLK, BLK, DMODEL)[perm].reshape(SQ, DMODEL).astype(bf)
    k_p = (
        K_ext.reshape(SKV_LOC // BLK, BLK, HQ * DH)[perm]
        .reshape(SKV_LOC, HQ * DH).astype(bf)
    )
    v_p = (
        V_ext.reshape(SKV_LOC // BLK, BLK, HQ * DH)[perm]
        .reshape(SKV_LOC, HQ * DH).astype(bf)
    )

    o, ml = pl.pallas_call(
        _attn_body,
        in_specs=[pl.BlockSpec(memory_space=pltpu.VMEM)] * 4,
        out_shape=[
            jax.ShapeDtypeStruct((SQ, DMODEL), jnp.bfloat16),
            jax.ShapeDtypeStruct((SQ, 2 * HQ), jnp.float32),
        ],
        out_specs=[pl.BlockSpec(memory_space=pltpu.VMEM)] * 2,
        compiler_params=pltpu.CompilerParams(vmem_limit_bytes=100 * 1024 * 1024),
    )(x_p, Wq.astype(bf), k_p, v_p)

    out = pl.pallas_call(
        _combine_body,
        in_specs=[pl.BlockSpec(memory_space=pltpu.VMEM)] * 3,
        out_shape=jax.ShapeDtypeStruct((SQ, DMODEL), jnp.float32),
        out_specs=pl.BlockSpec(memory_space=pltpu.VMEM),
        scratch_shapes=[
            pltpu.VMEM((SQ, DMODEL), jnp.float32),
            pltpu.VMEM((SQ, 2 * HQ), jnp.float32),
        ] + [
            pltpu.VMEM((7 * CH, HCOL), jnp.bfloat16),
            pltpu.VMEM((7 * CH, HCOL), jnp.bfloat16),
            pltpu.VMEM((7 * CH, HQ), jnp.float32),
            pltpu.VMEM((7 * CH, HQ), jnp.float32),
            pltpu.VMEM((SQ, HCOL), jnp.bfloat16),
        ] * 2 + [pltpu.SemaphoreType.DMA((3,))] * 12,
        compiler_params=pltpu.CompilerParams(
            collective_id=0, vmem_limit_bytes=100 * 1024 * 1024
        ),
    )(o, ml, Wo.astype(bf))

    return out.reshape(1, SQ, DMODEL)


# device time: 127517 ns/iter; 6.5924x vs baseline; 1.0816x over previous
import jax
import jax.numpy as jnp
from jax import lax
from jax.experimental import pallas as pl
from jax.experimental.pallas import tpu as pltpu

N_DEV = 8
SQ = 2048
DMODEL = 1024
HQ = 8
DH = 128
SKV_LOC = 2048
BLK = 64
NRES = 4
NJ = SQ // (BLK * NRES)
RROWS = SQ // NRES
CH = SQ // N_DEV
SCALE = 0.08838834764831843

PERM = [r + NRES * j for r in range(NRES) for j in range(NJ)]


def _attn_body(x_v, wq_ref, k_v, v_v, o_ref, l_ref):
    q_bf = jnp.dot(
        x_v[...], wq_ref[...], preferred_element_type=jnp.float32
    ).astype(jnp.bfloat16)
    for r in range(NRES):
        rs = pl.ds(r * RROWS, RROWS)
        lrs = []
        for hh in range(HQ):
            hs = pl.ds(hh * DH, DH)
            qr = q_bf[r * RROWS:(r + 1) * RROWS, hh * DH:(hh + 1) * DH]
            s = lax.dot_general(
                qr, k_v[rs, hs], (((1,), (1,)), ((), ())),
                preferred_element_type=jnp.float32,
            ) * SCALE
            p = jnp.exp(s)
            lr = jnp.sum(p, axis=1)
            orr = jnp.dot(
                p.astype(jnp.bfloat16), v_v[rs, hs],
                preferred_element_type=jnp.float32,
            )
            o_ref[rs, hs] = orr.astype(jnp.bfloat16)
            lrs.append(lr[:, None])
        l_ref[rs, 0:HQ] = jnp.concatenate(lrs, axis=1)


def _store_chunk_unpermuted(out_ref, chunk, c):
    for i in range(CH // BLK):
        p = c * (CH // BLK) + i
        blk = lax.rem(p, NJ) * NRES + lax.div(p, NJ)
        out_ref[pl.ds(blk * BLK, BLK), :] = chunk[i * BLK:(i + 1) * BLK, :]


def _store_half_chunk(out_ref, chunk, c, co):
    for i in range(CH // BLK):
        p = c * (CH // BLK) + i
        blk = lax.rem(p, NJ) * NRES + lax.div(p, NJ)
        out_ref[pl.ds(blk * BLK, BLK), pl.ds(co, HCOL)] = (
            chunk[i * BLK:(i + 1) * BLK, :].astype(jnp.float32)
        )


HCOL = DMODEL // 2
HH = HQ // 2

RS_MASKS = [[1, 3, 4], [4, 1, 3]]
RS_SPANS = [
    [[0, 3, 4, 7], [0, 4], [0]],
    [[0, 1, 3, 2], [0, 3], [0]],
]
RS_OFF = [0, 4 * CH, 6 * CH]
AG_MASKS = [[4, 3, 1], [3, 1, 4]]
AG_G = [[0, 4, 3, 7, 1, 5, 2, 6], [0, 3, 1, 2, 4, 7, 5, 6]]


def _combine_body(
    o_ref, l_ref, wo_ref, out_ref,
    acc_o, acc_l,
    o_snd0, o_rcv0, ml_snd0, ml_rcv0, ag0,
    o_snd1, o_rcv1, ml_snd1, ml_rcv1, ag1,
    o_ssem0, o_rsem0, ml_ssem0, ml_rsem0, ag_ssem0, ag_rsem0,
    o_ssem1, o_rsem1, ml_ssem1, ml_rsem1, ag_ssem1, ag_rsem1,
):
    my = lax.axis_index("i")
    trees = [
        (0, 0, o_snd0, o_rcv0, ml_snd0, ml_rcv0, ag0,
         o_ssem0, o_rsem0, ml_ssem0, ml_rsem0, ag_ssem0, ag_rsem0),
        (HCOL, HH, o_snd1, o_rcv1, ml_snd1, ml_rcv1, ag1,
         o_ssem1, o_rsem1, ml_ssem1, ml_rsem1, ag_ssem1, ag_rsem1),
    ]

    barrier = pltpu.get_barrier_semaphore()
    for mask in (1, 3, 4):
        pl.semaphore_signal(
            barrier, inc=1, device_id=(my ^ mask,),
            device_id_type=pl.DeviceIdType.MESH,
        )
    pl.semaphore_wait(barrier, 3)

    acc_o[...] = o_ref[...].astype(jnp.float32)
    acc_l[...] = l_ref[...]

    for k in range(3):
        off = RS_OFF[k]
        rdmas = []
        for d, (co, ho, o_snd, o_rcv, ml_snd, ml_rcv, _ag,
                o_ss, o_rs, ml_ss, ml_rs, _as, _ar) in enumerate(trees):
            mask = RS_MASKS[d][k]
            span = RS_SPANS[d][k]
            n = len(span)
            for idx, s in enumerate(span):
                src = pl.ds((my ^ (mask ^ s)) * CH, CH)
                dst = pl.ds(off + idx * CH, CH)
                o_snd[dst, :] = acc_o[src, pl.ds(co, HCOL)].astype(jnp.bfloat16)
                ml_snd[dst, :] = acc_l[src, ho:ho + HH]
            o_rdma = pltpu.make_async_remote_copy(
                src_ref=o_snd.at[pl.ds(off, n * CH), :],
                dst_ref=o_rcv.at[pl.ds(off, n * CH), :],
                send_sem=o_ss.at[k], recv_sem=o_rs.at[k],
                device_id=(my ^ mask,), device_id_type=pl.DeviceIdType.MESH,
            )
            ml_rdma = pltpu.make_async_remote_copy(
                src_ref=ml_snd.at[pl.ds(off, n * CH), :],
                dst_ref=ml_rcv.at[pl.ds(off, n * CH), :],
                send_sem=ml_ss.at[k], recv_sem=ml_rs.at[k],
                device_id=(my ^ mask,), device_id_type=pl.DeviceIdType.MESH,
            )
            o_rdma.start()
            ml_rdma.start()
            rdmas.append((o_rdma, ml_rdma))
        for d, (co, ho, o_snd, o_rcv, ml_snd, ml_rcv, *_r) in enumerate(trees):
            rdmas[d][0].wait()
            rdmas[d][1].wait()
            span = RS_SPANS[d][k]
            for idx, s in enumerate(span):
                rs = pl.ds((my ^ s) * CH, CH)
                rr = pl.ds(off + idx * CH, CH)
                acc_l[rs, ho:ho + HH] = (
                    acc_l[rs, ho:ho + HH] + ml_rcv[rr, 0:HH]
                )
                acc_o[rs, pl.ds(co, HCOL)] = (
                    acc_o[rs, pl.ds(co, HCOL)]
                    + o_rcv[rr, :].astype(jnp.float32)
                )

    qs = pl.ds(my * CH, CH)
    ctx = jnp.concatenate(
        [
            acc_o[qs, pl.ds(hh * DH, DH)] / acc_l[qs, hh:hh + 1]
            for hh in range(HQ)
        ],
        axis=1,
    )
    mine = jnp.dot(
        ctx.astype(jnp.bfloat16), wo_ref[...],
        preferred_element_type=jnp.float32,
    )
    _store_chunk_unpermuted(out_ref, mine, my)
    ag0[pl.ds(0, CH), :] = mine[:, 0:HCOL].astype(jnp.bfloat16)
    ag1[pl.ds(0, CH), :] = mine[:, HCOL:DMODEL].astype(jnp.bfloat16)

    for k in range(3):
        n = 1 << k
        rdmas = []
        for d, (co, ho, *_b) in enumerate(trees):
            ag = trees[d][6]
            ag_ss = trees[d][11]
            ag_rs = trees[d][12]
            mask = AG_MASKS[d][k]
            rdma = pltpu.make_async_remote_copy(
                src_ref=ag.at[pl.ds(0, n * CH), :],
                dst_ref=ag.at[pl.ds(n * CH, n * CH), :],
                send_sem=ag_ss.at[k], recv_sem=ag_rs.at[k],
                device_id=(my ^ mask,), device_id_type=pl.DeviceIdType.MESH,
            )
            rdma.start()
            rdmas.append(rdma)
        for d in range(2):
            rdmas[d].wait()
            ag = trees[d][6]
            co = trees[d][0]
            for j in range(n, 2 * n):
                _store_half_chunk(
                    out_ref, ag[pl.ds(j * CH, CH), :], my ^ AG_G[d][j], co
                )


def kernel(x, Wq, K_ext, V_ext, Wo):
    perm = jnp.asarray(PERM, dtype=jnp.int32)
    bf = jnp.bfloat16
    x_p = x.reshape(SQ // BLK, BLK, DMODEL)[perm].reshape(SQ, DMODEL).astype(bf)
    k_p = (
        K_ext.reshape(SKV_LOC // BLK, BLK, HQ * DH)[perm]
        .reshape(SKV_LOC, HQ * DH).astype(bf)
    )
    v_p = (
        V_ext.reshape(SKV_LOC // BLK, BLK, HQ * DH)[perm]
        .reshape(SKV_LOC, HQ * DH).astype(bf)
    )

    o, ml = pl.pallas_call(
        _attn_body,
        in_specs=[pl.BlockSpec(memory_space=pltpu.VMEM)] * 4,
        out_shape=[
            jax.ShapeDtypeStruct((SQ, DMODEL), jnp.bfloat16),
            jax.ShapeDtypeStruct((SQ, HQ), jnp.float32),
        ],
        out_specs=[pl.BlockSpec(memory_space=pltpu.VMEM)] * 2,
        compiler_params=pltpu.CompilerParams(vmem_limit_bytes=100 * 1024 * 1024),
    )(x_p, Wq.astype(bf), k_p, v_p)

    out = pl.pallas_call(
        _combine_body,
        in_specs=[pl.BlockSpec(memory_space=pltpu.VMEM)] * 3,
        out_shape=jax.ShapeDtypeStruct((SQ, DMODEL), jnp.float32),
        out_specs=pl.BlockSpec(memory_space=pltpu.VMEM),
        scratch_shapes=[
            pltpu.VMEM((SQ, DMODEL), jnp.float32),
            pltpu.VMEM((SQ, HQ), jnp.float32),
        ] + [
            pltpu.VMEM((7 * CH, HCOL), jnp.bfloat16),
            pltpu.VMEM((7 * CH, HCOL), jnp.bfloat16),
            pltpu.VMEM((7 * CH, HH), jnp.float32),
            pltpu.VMEM((7 * CH, HH), jnp.float32),
            pltpu.VMEM((SQ, HCOL), jnp.bfloat16),
        ] * 2 + [pltpu.SemaphoreType.DMA((3,))] * 12,
        compiler_params=pltpu.CompilerParams(
            collective_id=0, vmem_limit_bytes=100 * 1024 * 1024
        ),
    )(o, ml, Wo.astype(bf))

    return out.reshape(1, SQ, DMODEL)


# device time: 126689 ns/iter; 6.6355x vs baseline; 1.0065x over previous
import jax
import jax.numpy as jnp
from jax import lax
from jax.experimental import pallas as pl
from jax.experimental.pallas import tpu as pltpu

N_DEV = 8
SQ = 2048
DMODEL = 1024
HQ = 8
DH = 128
SKV_LOC = 2048
BLK = 64
NRES = 4
NJ = SQ // (BLK * NRES)
RROWS = SQ // NRES
CH = SQ // N_DEV
SCALE = 0.08838834764831843

PERM = [r + NRES * j for r in range(NRES) for j in range(NJ)]


def _attn_body(x_v, wq_ref, k_v, v_v, o_ref, l_ref):
    q_bf = jnp.dot(
        x_v[...], wq_ref[...], preferred_element_type=jnp.float32
    ).astype(jnp.bfloat16)
    for r in range(NRES):
        rs = pl.ds(r * RROWS, RROWS)
        lrs = []
        for hh in range(HQ):
            hs = pl.ds(hh * DH, DH)
            qr = q_bf[r * RROWS:(r + 1) * RROWS, hh * DH:(hh + 1) * DH]
            s = lax.dot_general(
                qr, k_v[rs, hs], (((1,), (1,)), ((), ())),
                preferred_element_type=jnp.float32,
            ) * SCALE
            p = jnp.exp(s)
            lr = jnp.sum(p, axis=1)
            orr = jnp.dot(
                p.astype(jnp.bfloat16), v_v[rs, hs],
                preferred_element_type=jnp.float32,
            )
            o_ref[rs, hs] = orr.astype(jnp.bfloat16)
            lrs.append(lr[:, None])
        l_ref[rs, 0:HQ] = jnp.concatenate(lrs, axis=1)


def _store_chunk_unpermuted(out_ref, chunk, c):
    for i in range(CH // BLK):
        p = c * (CH // BLK) + i
        blk = lax.rem(p, NJ) * NRES + lax.div(p, NJ)
        out_ref[pl.ds(blk * BLK, BLK), :] = chunk[i * BLK:(i + 1) * BLK, :]


def _store_half_chunk(out_ref, chunk, c, co):
    for i in range(CH // BLK):
        p = c * (CH // BLK) + i
        blk = lax.rem(p, NJ) * NRES + lax.div(p, NJ)
        out_ref[pl.ds(blk * BLK, BLK), pl.ds(co, HCOL)] = (
            chunk[i * BLK:(i + 1) * BLK, :].astype(jnp.float32)
        )


HCOL = DMODEL // 2
HH = HQ // 2

RS_MASKS = [[1, 3, 4], [4, 1, 3]]
RS_SPANS = [
    [[0, 3, 4, 7], [0, 4], [0]],
    [[0, 1, 3, 2], [0, 3], [0]],
]
RS_OFF = [0, 4 * CH, 6 * CH]
AG_MASKS = [[4, 3, 1], [3, 1, 4]]
AG_G = [[0, 4, 3, 7, 1, 5, 2, 6], [0, 3, 1, 2, 4, 7, 5, 6]]


def _combine_body(
    o_ref, l_ref, wo_ref, out_ref,
    acc_o, acc_l,
    o_snd0, o_rcv0, ml_snd0, ml_rcv0, ag0,
    o_snd1, o_rcv1, ml_snd1, ml_rcv1, ag1,
    o_ssem0, o_rsem0, ml_ssem0, ml_rsem0, ag_ssem0, ag_rsem0,
    o_ssem1, o_rsem1, ml_ssem1, ml_rsem1, ag_ssem1, ag_rsem1,
):
    my = lax.axis_index("i")
    trees = [
        (0, 0, o_snd0, o_rcv0, ml_snd0, ml_rcv0, ag0,
         o_ssem0, o_rsem0, ml_ssem0, ml_rsem0, ag_ssem0, ag_rsem0),
        (HCOL, HH, o_snd1, o_rcv1, ml_snd1, ml_rcv1, ag1,
         o_ssem1, o_rsem1, ml_ssem1, ml_rsem1, ag_ssem1, ag_rsem1),
    ]

    barrier = pltpu.get_barrier_semaphore()
    for mask in (1, 3, 4):
        pl.semaphore_signal(
            barrier, inc=1, device_id=(my ^ mask,),
            device_id_type=pl.DeviceIdType.MESH,
        )
    pl.semaphore_wait(barrier, 3)

    acc_o[...] = o_ref[...].astype(jnp.float32)
    acc_l[...] = l_ref[...]

    for k in range(3):
        off = RS_OFF[k]
        rdmas = []
        for d, (co, ho, o_snd, o_rcv, ml_snd, ml_rcv, _ag,
                o_ss, o_rs, ml_ss, ml_rs, _as, _ar) in enumerate(trees):
            mask = RS_MASKS[d][k]
            span = RS_SPANS[d][k]
            n = len(span)
            for idx, s in enumerate(span):
                src = pl.ds((my ^ (mask ^ s)) * CH, CH)
                dst = pl.ds(off + idx * CH, CH)
                o_snd[dst, :] = acc_o[src, pl.ds(co, HCOL)].astype(jnp.bfloat16)
                ml_snd[dst, :] = acc_l[src, ho:ho + HH]
            o_rdma = pltpu.make_async_remote_copy(
                src_ref=o_snd.at[pl.ds(off, n * CH), :],
                dst_ref=o_rcv.at[pl.ds(off, n * CH), :],
                send_sem=o_ss.at[k], recv_sem=o_rs.at[k],
                device_id=(my ^ mask,), device_id_type=pl.DeviceIdType.MESH,
            )
            ml_rdma = pltpu.make_async_remote_copy(
                src_ref=ml_snd.at[pl.ds(off, n * CH), :],
                dst_ref=ml_rcv.at[pl.ds(off, n * CH), :],
                send_sem=ml_ss.at[k], recv_sem=ml_rs.at[k],
                device_id=(my ^ mask,), device_id_type=pl.DeviceIdType.MESH,
            )
            o_rdma.start()
            ml_rdma.start()
            rdmas.append((o_rdma, ml_rdma))
        for d, (co, ho, o_snd, o_rcv, ml_snd, ml_rcv, *_r) in enumerate(trees):
            rdmas[d][0].wait()
            rdmas[d][1].wait()
            span = RS_SPANS[d][k]
            for idx, s in enumerate(span):
                rs = pl.ds((my ^ s) * CH, CH)
                rr = pl.ds(off + idx * CH, CH)
                acc_l[rs, ho:ho + HH] = (
                    acc_l[rs, ho:ho + HH] + ml_rcv[rr, 0:HH]
                )
                acc_o[rs, pl.ds(co, HCOL)] = (
                    acc_o[rs, pl.ds(co, HCOL)]
                    + o_rcv[rr, :].astype(jnp.float32)
                )

    qs = pl.ds(my * CH, CH)
    ctx = jnp.concatenate(
        [
            acc_o[qs, pl.ds(hh * DH, DH)] / acc_l[qs, hh:hh + 1]
            for hh in range(HQ)
        ],
        axis=1,
    )
    mine = jnp.dot(
        ctx.astype(jnp.bfloat16), wo_ref[...],
        preferred_element_type=jnp.float32,
    )
    _store_chunk_unpermuted(out_ref, mine, my)
    ag0[pl.ds(0, CH), :] = mine[:, 0:HCOL].astype(jnp.bfloat16)
    ag1[pl.ds(0, CH), :] = mine[:, HCOL:DMODEL].astype(jnp.bfloat16)

    for k in range(3):
        n = 1 << k
        rdmas = []
        for d, (co, ho, *_b) in enumerate(trees):
            ag = trees[d][6]
            ag_ss = trees[d][11]
            ag_rs = trees[d][12]
            mask = AG_MASKS[d][k]
            rdma = pltpu.make_async_remote_copy(
                src_ref=ag.at[pl.ds(0, n * CH), :],
                dst_ref=ag.at[pl.ds(n * CH, n * CH), :],
                send_sem=ag_ss.at[k], recv_sem=ag_rs.at[k],
                device_id=(my ^ mask,), device_id_type=pl.DeviceIdType.MESH,
            )
            rdma.start()
            rdmas.append(rdma)
        for d in range(2):
            rdmas[d].wait()
            ag = trees[d][6]
            co = trees[d][0]
            for j in range(n, 2 * n):
                _store_half_chunk(
                    out_ref, ag[pl.ds(j * CH, CH), :], my ^ AG_G[d][j], co
                )


def kernel(x, Wq, K_ext, V_ext, Wo):
    perm = jnp.asarray(PERM, dtype=jnp.int32)
    bf = jnp.bfloat16
    x_p = x.astype(bf).reshape(SQ // BLK, BLK, DMODEL)[perm].reshape(SQ, DMODEL)
    k_p = (
        K_ext.astype(bf).reshape(SKV_LOC // BLK, BLK, HQ * DH)[perm]
        .reshape(SKV_LOC, HQ * DH)
    )
    v_p = (
        V_ext.astype(bf).reshape(SKV_LOC // BLK, BLK, HQ * DH)[perm]
        .reshape(SKV_LOC, HQ * DH)
    )

    o, ml = pl.pallas_call(
        _attn_body,
        in_specs=[pl.BlockSpec(memory_space=pltpu.VMEM)] * 4,
        out_shape=[
            jax.ShapeDtypeStruct((SQ, DMODEL), jnp.bfloat16),
            jax.ShapeDtypeStruct((SQ, HQ), jnp.float32),
        ],
        out_specs=[pl.BlockSpec(memory_space=pltpu.VMEM)] * 2,
        compiler_params=pltpu.CompilerParams(vmem_limit_bytes=100 * 1024 * 1024),
    )(x_p, Wq.astype(bf), k_p, v_p)

    out = pl.pallas_call(
        _combine_body,
        in_specs=[pl.BlockSpec(memory_space=pltpu.VMEM)] * 3,
        out_shape=jax.ShapeDtypeStruct((SQ, DMODEL), jnp.float32),
        out_specs=pl.BlockSpec(memory_space=pltpu.VMEM),
        scratch_shapes=[
            pltpu.VMEM((SQ, DMODEL), jnp.float32),
            pltpu.VMEM((SQ, HQ), jnp.float32),
        ] + [
            pltpu.VMEM((7 * CH, HCOL), jnp.bfloat16),
            pltpu.VMEM((7 * CH, HCOL), jnp.bfloat16),
            pltpu.VMEM((7 * CH, HH), jnp.float32),
            pltpu.VMEM((7 * CH, HH), jnp.float32),
            pltpu.VMEM((SQ, HCOL), jnp.bfloat16),
        ] * 2 + [pltpu.SemaphoreType.DMA((3,))] * 12,
        compiler_params=pltpu.CompilerParams(
            collective_id=0, vmem_limit_bytes=100 * 1024 * 1024
        ),
    )(o, ml, Wo.astype(bf))

    return out.reshape(1, SQ, DMODEL)
